# baseline (speedup 1.0000x reference)
import functools

import jax
import jax.numpy as jnp
from jax import lax
from jax.experimental import pallas as pl
from jax.experimental.pallas import tpu as pltpu
from jax.experimental.pallas import tpu_sc as plsc

B = 16384
D = 32
V = 1000000
RPG = 128 // D
NC = 2
NS = 16
NW = NC * NS
B_PER_W = B // NW
CHUNK = 128
NCHUNK = B_PER_W // CHUNK
L = 16

_mesh = plsc.VectorSubcoreMesh(core_axis_name="c", subcore_axis_name="s")


@functools.partial(
    pl.kernel,
    mesh=_mesh,
    out_type=jax.ShapeDtypeStruct((B * D // 128, 128), jnp.float32),
    scratch_types=[
        pltpu.VMEM((NCHUNK, CHUNK), jnp.int32),
        pltpu.VMEM((NCHUNK, CHUNK), jnp.int32),
        pltpu.VMEM((B_PER_W, 128), jnp.float32),
        pltpu.VMEM((B_PER_W * D // 128, 128), jnp.float32),
        pltpu.SemaphoreType.DMA,
    ],
    compiler_params=pltpu.CompilerParams(needs_layout_passes=False),
)
def _gather_kernel(table_hbm, idx_hbm, out_hbm, idx_v, g_v, rows_v, out_v, sem):
    wid = lax.axis_index("s") * NC + lax.axis_index("c")
    pltpu.sync_copy(idx_hbm.at[wid], idx_v)
    for i in range(NCHUNK):
        for k in range(CHUNK // L):
            s = pl.ds(k * L, L)
            g_v[i, s] = jax.lax.shift_right_logical(idx_v[i, s], 2)
    copies = [
        pltpu.async_copy(
            table_hbm.at[g_v.at[i]],
            rows_v.at[pl.ds(i * CHUNK, CHUNK)],
            sem,
        )
        for i in range(NCHUNK)
    ]
    for c in copies:
        c.wait()
    iota = lax.iota(jnp.int32, L)
    for i in range(NCHUNK):
        for b in range(CHUNK // L):
            s = pl.ds(b * L, L)
            jv = iota + (i * CHUNK + b * L)
            rb = jax.lax.shift_left(idx_v[i, s] & 3, 5)
            tv = jax.lax.shift_right_logical(jv, 2)
            cb = jax.lax.shift_left(jv & 3, 5)

            def body(c, carry, jv=jv, rb=rb, tv=tv, cb=cb):
                vals = plsc.load_gather(rows_v, [jv, rb + c])
                plsc.store_scatter(out_v, [tv, cb + c], vals)
                return carry

            lax.fori_loop(0, D, body, 0)
    rows_out = B_PER_W * D // 128
    pltpu.sync_copy(out_v, out_hbm.at[pl.ds(wid * rows_out, rows_out)])


def kernel(node_embeddings, object_indices):
    table4 = node_embeddings.reshape(V // RPG, 128)
    idx = object_indices.astype(jnp.int32).reshape(NW, NCHUNK, CHUNK)
    out4 = _gather_kernel(table4, idx)
    return out4.reshape(B, D)


# speedup vs baseline: 4.3939x; 4.3939x over previous
import functools

import jax
import jax.numpy as jnp
from jax import lax
from jax.experimental import pallas as pl
from jax.experimental.pallas import tpu as pltpu
from jax.experimental.pallas import tpu_sc as plsc

B = 16384
D = 32
V = 1000000
NC = 2
NS = 16
NW = NC * NS
B_PER_W = B // NW
NBUF = 8
L = 16

_mesh = plsc.VectorSubcoreMesh(core_axis_name="c", subcore_axis_name="s")


@functools.partial(
    pl.kernel,
    mesh=_mesh,
    out_type=jax.ShapeDtypeStruct((D, B), jnp.float32),
    scratch_types=[
        pltpu.VMEM((B_PER_W + L,), jnp.int32),
        pltpu.VMEM((NBUF, D, 128), jnp.float32),
        pltpu.VMEM((D, B_PER_W), jnp.float32),
        [pltpu.SemaphoreType.DMA] * NBUF,
    ],
    compiler_params=pltpu.CompilerParams(needs_layout_passes=False),
)
def _gather_kernel(table_hbm, idx_hbm, out_hbm, idx_v, tiles_v, out_v, sems):
    wid = lax.axis_index("s") * NC + lax.axis_index("c")
    base = wid * B_PER_W
    pltpu.sync_copy(idx_hbm.at[pl.ds(base, B_PER_W)],
                    idx_v.at[pl.ds(0, B_PER_W)])

    def fire(j, b):
        i = idx_v[pl.ds(j, L)][0]
        off = pl.multiple_of(i & ~127, 128)
        pltpu.async_copy(
            table_hbm.at[:, pl.ds(off, 128)],
            tiles_v.at[b],
            sems[b],
        )

    for b in range(NBUF):
        fire(b, b)

    iota = lax.iota(jnp.int32, L)
    rows0 = iota
    rows1 = iota + L

    def body(g, carry):
        for b in range(NBUF):
            j = g * NBUF + b
            pltpu.make_async_copy(
                table_hbm.at[:, pl.ds(0, 128)], tiles_v.at[b], sems[b]
            ).wait()
            w = idx_v[pl.ds(j, L)][0] & 127
            col = jnp.full((L,), w, jnp.int32)
            jcol = jnp.full((L,), j, jnp.int32)
            vals0 = plsc.load_gather(tiles_v.at[b], [rows0, col])
            vals1 = plsc.load_gather(tiles_v.at[b], [rows1, col])
            plsc.store_scatter(out_v, [rows0, jcol], vals0)
            plsc.store_scatter(out_v, [rows1, jcol], vals1)
            @pl.when(g < B_PER_W // NBUF - 1)
            def _():
                fire(j + NBUF, b)
        return carry

    lax.fori_loop(0, B_PER_W // NBUF, body, 0)
    pltpu.sync_copy(out_v, out_hbm.at[:, pl.ds(base, B_PER_W)])


def kernel(node_embeddings, object_indices):
    table_t = node_embeddings.T
    idx = object_indices.astype(jnp.int32)
    out_t = _gather_kernel(table_t, idx)
    return out_t.T
